# baseline (speedup 1.0000x reference)
import functools

import jax
import jax.numpy as jnp
from jax import lax
from jax.experimental import pallas as pl
from jax.experimental.pallas import tpu as pltpu
from jax.experimental.pallas import tpu_sc as plsc

N_NODES_C = 10000
N_EDGES_C = 320000
D = 128

NC = 2
NS = 16
NW = NC * NS
EDGES_PER_TILE = N_EDGES_C // NW
CHUNK = 80
N_CHUNKS = EDGES_PER_TILE // CHUNK
NBUF = 3
ROWS_PER_TILE = 624
ROWS_TAIL = N_NODES_C - NS * ROWS_PER_TILE


def _matmul_body(x_ref, w_ref, o_ref):
    o_ref[...] = jnp.dot(x_ref[...], w_ref[...], preferred_element_type=jnp.float32)


def _tc_matmul(x, W):
    return pl.pallas_call(
        _matmul_body,
        grid=(10,),
        in_specs=[
            pl.BlockSpec((1000, D), lambda i: (i, 0)),
            pl.BlockSpec((D, D), lambda i: (0, 0)),
        ],
        out_specs=pl.BlockSpec((1000, D), lambda i: (i, 0)),
        out_shape=jax.ShapeDtypeStruct((N_NODES_C, D), jnp.float32),
    )(x, W)


def _combine_body(a_ref, b_ref, o_ref):
    o_ref[...] = jnp.maximum(a_ref[...] + b_ref[...], 0.0)


def _tc_combine(partials):
    return pl.pallas_call(
        _combine_body,
        grid=(10,),
        in_specs=[
            pl.BlockSpec((1000, D), lambda i: (i, 0)),
            pl.BlockSpec((1000, D), lambda i: (i + 10, 0)),
        ],
        out_specs=pl.BlockSpec((1000, D), lambda i: (i, 0)),
        out_shape=jax.ShapeDtypeStruct((N_NODES_C, D), jnp.float32),
    )(partials, partials)


def _sc_aggregate(pre_sup, edata, zeros):
    mesh = plsc.VectorSubcoreMesh(core_axis_name="c", subcore_axis_name="s")

    @functools.partial(
        pl.kernel,
        out_type=jax.ShapeDtypeStruct((NC * N_NODES_C, D), jnp.float32),
        mesh=mesh,
        compiler_params=pltpu.CompilerParams(needs_layout_passes=False),
        scratch_types=[
            pltpu.VMEM_SHARED((N_NODES_C, D), jnp.float32),
            pltpu.VMEM((NBUF, 3, CHUNK), jnp.int32),
            pltpu.VMEM((NBUF, CHUNK, D), jnp.float32),
            pltpu.SemaphoreType.DMA((NBUF,)),
            pltpu.SemaphoreType.DMA((NBUF,)),
            pltpu.SemaphoreType.DMA((NBUF,)),
        ],
    )
    def agg(pre_hbm, ed_hbm, z_hbm, out_hbm,
            acc, ebuf, rows_v, sem_e, sem_g, sem_s):
        c = lax.axis_index("c")
        s = lax.axis_index("s")
        wid = s * NC + c

        pltpu.sync_copy(z_hbm, acc.at[pl.ds(s * ROWS_PER_TILE, ROWS_PER_TILE)])

        @pl.when(s == NS - 1)
        def _zero_tail():
            pltpu.sync_copy(
                z_hbm.at[pl.ds(0, ROWS_TAIL)],
                acc.at[pl.ds(NS * ROWS_PER_TILE, ROWS_TAIL)],
            )

        plsc.subcore_barrier()

        def issue_edata(b, ch):
            pltpu.async_copy(ed_hbm.at[wid, ch], ebuf.at[b], sem_e.at[b])

        def wait_edata(b, ch):
            pltpu.make_async_copy(
                ed_hbm.at[wid, ch], ebuf.at[b], sem_e.at[b]
            ).wait()

        def issue_gather(b, ch):
            h = CHUNK // 2
            pltpu.async_copy(
                pre_hbm.at[ebuf.at[b, 0, pl.ds(0, h)]],
                rows_v.at[b, pl.ds(0, h)], sem_g.at[b])
            pltpu.async_copy(
                pre_hbm.at[ebuf.at[b, 0, pl.ds(h, h)]],
                rows_v.at[b, pl.ds(h, h)], sem_g.at[b])

        def wait_gather(b, ch):
            pltpu.make_async_copy(
                pre_hbm.at[ebuf.at[b, 0]], rows_v.at[b], sem_g.at[b]
            ).wait()

        def issue_scatter(b, ch):
            pltpu.async_copy(
                rows_v.at[b], acc.at[ebuf.at[b, 1]], sem_s.at[b], add=True
            )

        def wait_scatter(b, ch):
            pltpu.make_async_copy(
                rows_v.at[b], acc.at[ebuf.at[b, 1]], sem_s.at[b]
            ).wait()

        issue_edata(0, 0)
        wait_edata(0, 0)
        issue_gather(0, 0)
        issue_edata(1, 1)

        def chunk_step(ch, b):
            bn = (b + 1) % NBUF
            b2 = (b + 2) % NBUF
            wait_gather(b, ch)

            @pl.when(ch + 1 < N_CHUNKS)
            def _g():
                wait_edata(bn, ch + 1)
                issue_gather(bn, ch + 1)

            def scale4(t, carry):
                for u in range(4):
                    e = t * 4 + u
                    wi = plsc.load_gather(
                        ebuf.at[b, 2], [jnp.full((16,), e, jnp.int32)]
                    )
                    ws = plsc.bitcast(wi, jnp.float32)
                    for i in range(D // 16):
                        sl = pl.ds(i * 16, 16)
                        rows_v[b, e, sl] = rows_v[b, e, sl] * ws
                return carry

            lax.fori_loop(0, CHUNK // 4, scale4, None)

            @pl.when(jnp.logical_and(ch + 2 < N_CHUNKS, ch >= 1))
            def _ws2():
                wait_scatter(b2, ch - 1)

            @pl.when(ch + 2 < N_CHUNKS)
            def _e():
                issue_edata(b2, ch + 2)

            issue_scatter(b, ch)

        def outer(k, carry):
            for j in range(NBUF):
                chunk_step(NBUF * k + j, j)
            return carry

        n_full = (N_CHUNKS // NBUF) * NBUF
        lax.fori_loop(0, N_CHUNKS // NBUF, outer, None)
        for ch in range(n_full, N_CHUNKS):
            chunk_step(ch, ch % NBUF)

        for ch in range(N_CHUNKS - NBUF, N_CHUNKS):
            wait_scatter(ch % NBUF, ch)

        plsc.subcore_barrier()

        pltpu.sync_copy(
            acc.at[pl.ds(s * ROWS_PER_TILE, ROWS_PER_TILE)],
            out_hbm.at[pl.ds(c * N_NODES_C + s * ROWS_PER_TILE, ROWS_PER_TILE)],
        )

        @pl.when(s == NS - 1)
        def _write_tail():
            pltpu.sync_copy(
                acc.at[pl.ds(NS * ROWS_PER_TILE, ROWS_TAIL)],
                out_hbm.at[pl.ds(c * N_NODES_C + NS * ROWS_PER_TILE, ROWS_TAIL)],
            )

    return agg(pre_sup, edata, zeros)


def kernel(x, edge_index, edge_weight, W):
    src = edge_index[0].astype(jnp.int32).reshape(NW, N_CHUNKS, 1, CHUNK)
    dst = edge_index[1].astype(jnp.int32).reshape(NW, N_CHUNKS, 1, CHUNK)
    wbits = lax.bitcast_convert_type(
        edge_weight.astype(jnp.float32), jnp.int32
    ).reshape(NW, N_CHUNKS, 1, CHUNK)
    edata = jnp.concatenate([src, dst, wbits], axis=2)
    zeros = jnp.zeros((ROWS_PER_TILE, D), jnp.float32)

    pre_sup = _tc_matmul(x, W)
    partials = _sc_aggregate(pre_sup, edata, zeros)
    return _tc_combine(partials)


# speedup vs baseline: 1.1711x; 1.1711x over previous
import functools

import jax
import jax.numpy as jnp
from jax import lax
from jax.experimental import pallas as pl
from jax.experimental.pallas import tpu as pltpu
from jax.experimental.pallas import tpu_sc as plsc

N_NODES_C = 10000
N_EDGES_C = 320000
D = 128

NC = 2
NS = 16
NW = NC * NS
EDGES_PER_TILE = N_EDGES_C // NW
CHUNK = 80
N_CHUNKS = EDGES_PER_TILE // CHUNK
NBUF = 4
ROWS_PER_TILE = 624
ROWS_TAIL = N_NODES_C - NS * ROWS_PER_TILE


def _matmul_body(x_ref, w_ref, o_ref):
    o_ref[...] = jnp.dot(x_ref[...], w_ref[...], preferred_element_type=jnp.float32)


def _tc_matmul(x, W):
    return pl.pallas_call(
        _matmul_body,
        grid=(10,),
        in_specs=[
            pl.BlockSpec((1000, D), lambda i: (i, 0)),
            pl.BlockSpec((D, D), lambda i: (0, 0)),
        ],
        out_specs=pl.BlockSpec((1000, D), lambda i: (i, 0)),
        out_shape=jax.ShapeDtypeStruct((N_NODES_C, D), jnp.float32),
    )(x, W)


def _combine_body(a_ref, b_ref, o_ref):
    o_ref[...] = jnp.maximum(a_ref[...] + b_ref[...], 0.0)


def _tc_combine(partials):
    return pl.pallas_call(
        _combine_body,
        grid=(10,),
        in_specs=[
            pl.BlockSpec((1000, D), lambda i: (i, 0)),
            pl.BlockSpec((1000, D), lambda i: (i + 10, 0)),
        ],
        out_specs=pl.BlockSpec((1000, D), lambda i: (i, 0)),
        out_shape=jax.ShapeDtypeStruct((N_NODES_C, D), jnp.float32),
    )(partials, partials)


def _sc_aggregate(pre_sup, edata, zeros):
    mesh = plsc.VectorSubcoreMesh(core_axis_name="c", subcore_axis_name="s")

    @functools.partial(
        pl.kernel,
        out_type=jax.ShapeDtypeStruct((NC * N_NODES_C, D), jnp.float32),
        mesh=mesh,
        compiler_params=pltpu.CompilerParams(needs_layout_passes=False),
        scratch_types=[
            pltpu.VMEM_SHARED((N_NODES_C, D), jnp.float32),
            pltpu.VMEM((NBUF, 3, CHUNK), jnp.int32),
            pltpu.VMEM((NBUF, CHUNK, D), jnp.float32),
            pltpu.SemaphoreType.DMA((NBUF,)),
            pltpu.SemaphoreType.DMA((NBUF,)),
            pltpu.SemaphoreType.DMA((NBUF,)),
        ],
    )
    def agg(pre_hbm, ed_hbm, z_hbm, out_hbm,
            acc, ebuf, rows_v, sem_e, sem_g, sem_s):
        c = lax.axis_index("c")
        s = lax.axis_index("s")
        wid = s * NC + c

        pltpu.sync_copy(z_hbm, acc.at[pl.ds(s * ROWS_PER_TILE, ROWS_PER_TILE)])

        @pl.when(s == NS - 1)
        def _zero_tail():
            pltpu.sync_copy(
                z_hbm.at[pl.ds(0, ROWS_TAIL)],
                acc.at[pl.ds(NS * ROWS_PER_TILE, ROWS_TAIL)],
            )

        plsc.subcore_barrier()

        def issue_edata(b, ch):
            pltpu.async_copy(ed_hbm.at[wid, ch], ebuf.at[b], sem_e.at[b])

        def wait_edata(b, ch):
            pltpu.make_async_copy(
                ed_hbm.at[wid, ch], ebuf.at[b], sem_e.at[b]
            ).wait()

        def issue_gather(b, ch):
            pltpu.async_copy(pre_hbm.at[ebuf.at[b, 0]], rows_v.at[b], sem_g.at[b])

        def wait_gather(b, ch):
            pltpu.make_async_copy(
                pre_hbm.at[ebuf.at[b, 0]], rows_v.at[b], sem_g.at[b]
            ).wait()

        def issue_scatter(b, ch):
            pltpu.async_copy(
                rows_v.at[b], acc.at[ebuf.at[b, 1]], sem_s.at[b], add=True
            )

        def wait_scatter(b, ch):
            pltpu.make_async_copy(
                rows_v.at[b], acc.at[ebuf.at[b, 1]], sem_s.at[b]
            ).wait()

        issue_edata(0, 0)
        issue_edata(1, 1)
        issue_edata(2, 2)
        wait_edata(0, 0)
        issue_gather(0, 0)
        wait_edata(1, 1)
        issue_gather(1, 1)

        def chunk_step(ch, b):
            b2 = (b + 2) % NBUF
            b3 = (b + 3) % NBUF
            wait_gather(b, ch)

            def scale4(t, carry):
                for u in range(4):
                    e = t * 4 + u
                    wi = plsc.load_gather(
                        ebuf.at[b, 2], [jnp.full((16,), e, jnp.int32)]
                    )
                    ws = plsc.bitcast(wi, jnp.float32)
                    for i in range(D // 16):
                        sl = pl.ds(i * 16, 16)
                        rows_v[b, e, sl] = rows_v[b, e, sl] * ws
                return carry

            lax.fori_loop(0, CHUNK // 4, scale4, None)

            @pl.when(ch + 2 < N_CHUNKS)
            def _g():
                wait_edata(b2, ch + 2)
                issue_gather(b2, ch + 2)

            @pl.when(jnp.logical_and(ch + 3 < N_CHUNKS, ch >= 1))
            def _ws():
                wait_scatter(b3, ch - 1)

            @pl.when(ch + 3 < N_CHUNKS)
            def _e():
                issue_edata(b3, ch + 3)

            issue_scatter(b, ch)

        def outer(k, carry):
            for j in range(NBUF):
                chunk_step(NBUF * k + j, j)
            return carry

        n_full = (N_CHUNKS // NBUF) * NBUF
        lax.fori_loop(0, N_CHUNKS // NBUF, outer, None)
        for ch in range(n_full, N_CHUNKS):
            chunk_step(ch, ch % NBUF)

        for ch in range(N_CHUNKS - NBUF, N_CHUNKS):
            wait_scatter(ch % NBUF, ch)

        plsc.subcore_barrier()

        pltpu.sync_copy(
            acc.at[pl.ds(s * ROWS_PER_TILE, ROWS_PER_TILE)],
            out_hbm.at[pl.ds(c * N_NODES_C + s * ROWS_PER_TILE, ROWS_PER_TILE)],
        )

        @pl.when(s == NS - 1)
        def _write_tail():
            pltpu.sync_copy(
                acc.at[pl.ds(NS * ROWS_PER_TILE, ROWS_TAIL)],
                out_hbm.at[pl.ds(c * N_NODES_C + NS * ROWS_PER_TILE, ROWS_TAIL)],
            )

    return agg(pre_sup, edata, zeros)


def kernel(x, edge_index, edge_weight, W):
    src = edge_index[0].astype(jnp.int32).reshape(NW, N_CHUNKS, 1, CHUNK)
    dst = edge_index[1].astype(jnp.int32).reshape(NW, N_CHUNKS, 1, CHUNK)
    wbits = lax.bitcast_convert_type(
        edge_weight.astype(jnp.float32), jnp.int32
    ).reshape(NW, N_CHUNKS, 1, CHUNK)
    edata = jnp.concatenate([src, dst, wbits], axis=2)
    zeros = jnp.zeros((ROWS_PER_TILE, D), jnp.float32)

    pre_sup = _tc_matmul(x, W)
    partials = _sc_aggregate(pre_sup, edata, zeros)
    return _tc_combine(partials)


# speedup vs baseline: 1.4116x; 1.2053x over previous
import functools

import jax
import jax.numpy as jnp
from jax import lax
from jax.experimental import pallas as pl
from jax.experimental.pallas import tpu as pltpu
from jax.experimental.pallas import tpu_sc as plsc

N_NODES_C = 10000
N_EDGES_C = 320000
D = 128

NC = 2
NS = 16
NW = NC * NS
EDGES_PER_TILE = N_EDGES_C // NW
CHUNK = 80
N_CHUNKS = EDGES_PER_TILE // CHUNK
NBUF = 4
ROWS_PER_TILE = 624
ROWS_TAIL = N_NODES_C - NS * ROWS_PER_TILE


def _matmul_body(x_ref, w_ref, o_ref):
    o_ref[...] = jnp.dot(x_ref[...], w_ref[...], preferred_element_type=jnp.float32)


def _tc_matmul(x, W):
    return pl.pallas_call(
        _matmul_body,
        grid=(10,),
        in_specs=[
            pl.BlockSpec((1000, D), lambda i: (i, 0)),
            pl.BlockSpec((D, D), lambda i: (0, 0)),
        ],
        out_specs=pl.BlockSpec((1000, D), lambda i: (i, 0)),
        out_shape=jax.ShapeDtypeStruct((N_NODES_C, D), jnp.float32),
    )(x, W)


def _combine_body(a_ref, b_ref, o_ref):
    o_ref[...] = jnp.maximum(a_ref[...] + b_ref[...], 0.0)


def _tc_combine(partials):
    return pl.pallas_call(
        _combine_body,
        grid=(10,),
        in_specs=[
            pl.BlockSpec((1000, D), lambda i: (i, 0)),
            pl.BlockSpec((1000, D), lambda i: (i + 10, 0)),
        ],
        out_specs=pl.BlockSpec((1000, D), lambda i: (i, 0)),
        out_shape=jax.ShapeDtypeStruct((N_NODES_C, D), jnp.float32),
    )(partials, partials)


def _sc_aggregate(pre_sup, ei_flat, w_flat, zeros):
    mesh = plsc.VectorSubcoreMesh(core_axis_name="c", subcore_axis_name="s")

    @functools.partial(
        pl.kernel,
        out_type=jax.ShapeDtypeStruct((NC * N_NODES_C, D), jnp.float32),
        mesh=mesh,
        compiler_params=pltpu.CompilerParams(needs_layout_passes=False),
        scratch_types=[
            pltpu.VMEM_SHARED((N_NODES_C, D), jnp.float32),
            pltpu.VMEM((NBUF, CHUNK), jnp.int32),
            pltpu.VMEM((NBUF, CHUNK), jnp.int32),
            pltpu.VMEM((NBUF, CHUNK), jnp.float32),
            pltpu.VMEM((NBUF, CHUNK, D), jnp.float32),
            pltpu.SemaphoreType.DMA((NBUF,)),
            pltpu.SemaphoreType.DMA((NBUF,)),
            pltpu.SemaphoreType.DMA((NBUF,)),
        ],
    )
    def agg(pre_hbm, ei_hbm, w_hbm, z_hbm, out_hbm,
            acc, sbuf, dbuf, wbuf, rows_v, sem_e, sem_g, sem_s):
        c = lax.axis_index("c")
        s = lax.axis_index("s")
        wid = s * NC + c

        pltpu.sync_copy(z_hbm, acc.at[pl.ds(s * ROWS_PER_TILE, ROWS_PER_TILE)])

        @pl.when(s == NS - 1)
        def _zero_tail():
            pltpu.sync_copy(
                z_hbm.at[pl.ds(0, ROWS_TAIL)],
                acc.at[pl.ds(NS * ROWS_PER_TILE, ROWS_TAIL)],
            )

        plsc.subcore_barrier()

        ebase0 = wid * EDGES_PER_TILE

        def issue_edata(b, ch):
            e0 = ebase0 + ch * CHUNK
            pltpu.async_copy(
                ei_hbm.at[pl.ds(e0, CHUNK)], sbuf.at[b], sem_e.at[b])
            pltpu.async_copy(
                ei_hbm.at[pl.ds(N_EDGES_C + e0, CHUNK)], dbuf.at[b], sem_e.at[b])
            pltpu.async_copy(
                w_hbm.at[pl.ds(e0, CHUNK)], wbuf.at[b], sem_e.at[b])

        def wait_edata(b, ch):
            e0 = ebase0 + ch * CHUNK
            pltpu.make_async_copy(
                ei_hbm.at[pl.ds(e0, CHUNK)], sbuf.at[b], sem_e.at[b]).wait()
            pltpu.make_async_copy(
                ei_hbm.at[pl.ds(N_EDGES_C + e0, CHUNK)], dbuf.at[b],
                sem_e.at[b]).wait()
            pltpu.make_async_copy(
                w_hbm.at[pl.ds(e0, CHUNK)], wbuf.at[b], sem_e.at[b]).wait()

        def issue_gather(b, ch):
            pltpu.async_copy(pre_hbm.at[sbuf.at[b]], rows_v.at[b], sem_g.at[b])

        def wait_gather(b, ch):
            pltpu.make_async_copy(
                pre_hbm.at[sbuf.at[b]], rows_v.at[b], sem_g.at[b]
            ).wait()

        def issue_scatter(b, ch):
            pltpu.async_copy(
                rows_v.at[b], acc.at[dbuf.at[b]], sem_s.at[b], add=True
            )

        def wait_scatter(b, ch):
            pltpu.make_async_copy(
                rows_v.at[b], acc.at[dbuf.at[b]], sem_s.at[b]
            ).wait()

        issue_edata(0, 0)
        issue_edata(1, 1)
        issue_edata(2, 2)
        wait_edata(0, 0)
        issue_gather(0, 0)
        wait_edata(1, 1)
        issue_gather(1, 1)

        def chunk_step(ch, b):
            b2 = (b + 2) % NBUF
            b3 = (b + 3) % NBUF
            wait_gather(b, ch)

            def scale4(t, carry):
                for u in range(4):
                    e = t * 4 + u
                    ws = plsc.load_gather(
                        wbuf.at[b], [jnp.full((16,), e, jnp.int32)]
                    )
                    for i in range(D // 16):
                        sl = pl.ds(i * 16, 16)
                        rows_v[b, e, sl] = rows_v[b, e, sl] * ws
                return carry

            lax.fori_loop(0, CHUNK // 4, scale4, None)

            @pl.when(ch + 2 < N_CHUNKS)
            def _g():
                wait_edata(b2, ch + 2)
                issue_gather(b2, ch + 2)

            @pl.when(jnp.logical_and(ch + 3 < N_CHUNKS, ch >= 1))
            def _ws():
                wait_scatter(b3, ch - 1)

            @pl.when(ch + 3 < N_CHUNKS)
            def _e():
                issue_edata(b3, ch + 3)

            issue_scatter(b, ch)

        def outer(k, carry):
            for j in range(NBUF):
                chunk_step(NBUF * k + j, j)
            return carry

        n_full = (N_CHUNKS // NBUF) * NBUF
        lax.fori_loop(0, N_CHUNKS // NBUF, outer, None)
        for ch in range(n_full, N_CHUNKS):
            chunk_step(ch, ch % NBUF)

        for ch in range(N_CHUNKS - NBUF, N_CHUNKS):
            wait_scatter(ch % NBUF, ch)

        plsc.subcore_barrier()

        pltpu.sync_copy(
            acc.at[pl.ds(s * ROWS_PER_TILE, ROWS_PER_TILE)],
            out_hbm.at[pl.ds(c * N_NODES_C + s * ROWS_PER_TILE, ROWS_PER_TILE)],
        )

        @pl.when(s == NS - 1)
        def _write_tail():
            pltpu.sync_copy(
                acc.at[pl.ds(NS * ROWS_PER_TILE, ROWS_TAIL)],
                out_hbm.at[pl.ds(c * N_NODES_C + NS * ROWS_PER_TILE, ROWS_TAIL)],
            )

    return agg(pre_sup, ei_flat, w_flat, zeros)


def kernel(x, edge_index, edge_weight, W):
    ei_flat = edge_index.astype(jnp.int32).reshape(2 * N_EDGES_C)
    w_flat = edge_weight.astype(jnp.float32)
    zeros = jnp.zeros((ROWS_PER_TILE, D), jnp.float32)

    pre_sup = _tc_matmul(x, W)
    partials = _sc_aggregate(pre_sup, ei_flat, w_flat, zeros)
    return _tc_combine(partials)
